# baseline (speedup 1.0000x reference)
import functools

import jax
import jax.numpy as jnp
from jax import lax
from jax.experimental import pallas as pl
from jax.experimental.pallas import tpu as pltpu
from jax.experimental.pallas import tpu_sc as plsc

Q = 1024
D = 128
K = 100000
NN = 32

BK = 2048
CW = 128
NKT = (K + BK - 1) // BK
K_PAD = NKT * BK

_BIG = 3.0e38
_BIGI = 2**31 - 1


def _topk_kernel(qt_ref, tp_ref, oidx_ref, rv_ref, ri_ref, s_ref):
    kt = pl.program_id(0)

    @pl.when(kt == 0)
    def _init():
        rv_ref[...] = jnp.full((NN, Q), _BIG, jnp.float32)
        ri_ref[...] = jnp.full((NN, Q), _BIGI, jnp.int32)

    tpb = tp_ref[...]
    qt = qt_ref[...]
    t2 = jnp.sum(tpb * tpb, axis=1, keepdims=True)
    s = t2 - 2.0 * jnp.dot(tpb, qt, preferred_element_type=jnp.float32)

    row = lax.broadcasted_iota(jnp.int32, (BK, Q), 0)
    s = jnp.where(row + kt * BK >= K, _BIG, s)
    s_ref[...] = s

    slot = lax.broadcasted_iota(jnp.int32, (NN, Q), 0)
    crow = lax.broadcasted_iota(jnp.int32, (CW, Q), 0)

    for c in range(BK // CW):
        base = c * CW + kt * BK
        tau = rv_ref[NN - 1:NN, :]
        itau = ri_ref[NN - 1:NN, :]
        sc = s_ref[c * CW:(c + 1) * CW, :]
        beats = (sc < tau) | ((sc == tau) & (crow + base < itau))
        cnt = jnp.sum(beats.astype(jnp.int32), axis=0, keepdims=True)
        need = jnp.max(jnp.minimum(cnt, NN))

        def body(j, carry):
            sv = s_ref[c * CW:(c + 1) * CW, :]
            m = jnp.min(sv, axis=0, keepdims=True)
            sel = jnp.min(jnp.where(sv == m, crow, _BIGI), axis=0,
                          keepdims=True)
            s_ref[c * CW:(c + 1) * CW, :] = jnp.where(crow == sel, _BIG, sv)
            gsel = sel + base
            rv = rv_ref[...]
            ri = ri_ref[...]
            before = (rv < m) | ((rv == m) & (ri < gsel))
            pos = jnp.sum(before.astype(jnp.int32), axis=0, keepdims=True)
            rvs = pltpu.roll(rv, 1, axis=0)
            ris = pltpu.roll(ri, 1, axis=0)
            rv_ref[...] = jnp.where(slot < pos, rv,
                                    jnp.where(slot == pos, m, rvs))
            ri_ref[...] = jnp.where(slot < pos, ri,
                                    jnp.where(slot == pos, gsel, ris))
            return carry

        lax.fori_loop(0, need, body, 0)

    @pl.when(kt == NKT - 1)
    def _out():
        oidx_ref[...] = ri_ref[...]


def _topk_indices(qt, tp_pad):
    return pl.pallas_call(
        _topk_kernel,
        grid=(NKT,),
        in_specs=[
            pl.BlockSpec((D, Q), lambda k: (0, 0)),
            pl.BlockSpec((BK, D), lambda k: (k, 0)),
        ],
        out_specs=pl.BlockSpec((NN, Q), lambda k: (0, 0)),
        out_shape=jax.ShapeDtypeStruct((NN, Q), jnp.int32),
        scratch_shapes=[
            pltpu.VMEM((NN, Q), jnp.float32),
            pltpu.VMEM((NN, Q), jnp.int32),
            pltpu.VMEM((BK, Q), jnp.float32),
        ],
    )(qt, tp_pad)


NC = 2
NS = 16
NW = NC * NS
QPW = Q // NW


def _gather_mean(target_data, idx_flat):
    mesh = plsc.VectorSubcoreMesh(
        core_axis_name="c", subcore_axis_name="s", num_cores=NC,
        num_subcores=NS)

    @functools.partial(
        pl.kernel,
        out_type=jax.ShapeDtypeStruct((Q, D), jnp.float32),
        mesh=mesh,
        scratch_types=[
            pltpu.VMEM((QPW * NN,), jnp.int32),
            pltpu.VMEM((NN, D), jnp.float32),
            pltpu.VMEM((QPW, D), jnp.float32),
            pltpu.SemaphoreType.DMA,
        ],
    )
    def sc_kernel(table_hbm, idx_hbm, out_hbm, idx_v, rows_v, ostage_v, sem):
        wid = lax.axis_index("s") * NC + lax.axis_index("c")
        qbase = wid * QPW
        pltpu.sync_copy(idx_hbm.at[pl.ds(qbase * NN, QPW * NN)], idx_v)

        def per_query(qi, carry):
            off = pl.multiple_of(qi * NN, 8)
            pltpu.async_copy(
                table_hbm.at[idx_v.at[pl.ds(off, NN)]], rows_v, sem).wait()
            for c in range(D // 16):
                def body(r, acc):
                    return acc + rows_v[r, pl.ds(c * 16, 16)]
                acc = lax.fori_loop(0, NN, body, jnp.zeros((16,), jnp.float32))
                ostage_v[qi, pl.ds(c * 16, 16)] = acc * (1.0 / NN)
            return carry

        lax.fori_loop(0, QPW, per_query, 0)
        pltpu.sync_copy(ostage_v, out_hbm.at[pl.ds(qbase, QPW)])

    return sc_kernel(target_data, idx_flat)


def kernel(inputs, target_data):
    tp_pad = jnp.pad(target_data, ((0, K_PAD - K), (0, 0)))
    idx = _topk_indices(inputs.T, tp_pad)
    return _gather_mean(target_data, idx.T.reshape(-1))


# speedup vs baseline: 1.0472x; 1.0472x over previous
import functools

import jax
import jax.numpy as jnp
from jax import lax
from jax.experimental import pallas as pl
from jax.experimental.pallas import tpu as pltpu
from jax.experimental.pallas import tpu_sc as plsc

Q = 1024
D = 128
K = 100000
NN = 32

BK = 2048
CW = 256
NKT = (K + BK - 1) // BK
K_PAD = NKT * BK

_BIG = 3.0e38
_BIGI = 2**31 - 1


def _topk_kernel(qt_ref, tp_ref, oidx_ref, rv_ref, ri_ref, s_ref):
    kt = pl.program_id(0)

    @pl.when(kt == 0)
    def _init():
        rv_ref[...] = jnp.full((NN, Q), _BIG, jnp.float32)
        ri_ref[...] = jnp.full((NN, Q), _BIGI, jnp.int32)

    tpb = tp_ref[...]
    qt = qt_ref[...]
    t2 = jnp.sum(tpb * tpb, axis=1, keepdims=True)
    s = t2 - 2.0 * jnp.dot(tpb, qt, preferred_element_type=jnp.float32)

    row = lax.broadcasted_iota(jnp.int32, (BK, Q), 0)
    s = jnp.where(row + kt * BK >= K, _BIG, s)
    s_ref[...] = s

    slot = lax.broadcasted_iota(jnp.int32, (NN, Q), 0)
    crow = lax.broadcasted_iota(jnp.int32, (CW, Q), 0)

    for c in range(BK // CW):
        base = c * CW + kt * BK
        tau = rv_ref[NN - 1:NN, :]
        itau = ri_ref[NN - 1:NN, :]
        sc = s_ref[c * CW:(c + 1) * CW, :]
        beats = (sc < tau) | ((sc == tau) & (crow + base < itau))
        cnt = jnp.sum(beats.astype(jnp.int32), axis=0, keepdims=True)
        need = jnp.max(jnp.minimum(cnt, NN))

        def body(j, carry):
            sv = s_ref[c * CW:(c + 1) * CW, :]
            m = jnp.min(sv, axis=0, keepdims=True)
            sel = jnp.min(jnp.where(sv == m, crow, _BIGI), axis=0,
                          keepdims=True)
            s_ref[c * CW:(c + 1) * CW, :] = jnp.where(crow == sel, _BIG, sv)
            gsel = sel + base
            rv = rv_ref[...]
            ri = ri_ref[...]
            before = (rv < m) | ((rv == m) & (ri < gsel))
            pos = jnp.sum(before.astype(jnp.int32), axis=0, keepdims=True)
            rvs = pltpu.roll(rv, 1, axis=0)
            ris = pltpu.roll(ri, 1, axis=0)
            rv_ref[...] = jnp.where(slot < pos, rv,
                                    jnp.where(slot == pos, m, rvs))
            ri_ref[...] = jnp.where(slot < pos, ri,
                                    jnp.where(slot == pos, gsel, ris))
            return carry

        lax.fori_loop(0, need, body, 0)

    @pl.when(kt == NKT - 1)
    def _out():
        oidx_ref[...] = ri_ref[...]


def _topk_indices(qt, tp_pad):
    return pl.pallas_call(
        _topk_kernel,
        grid=(NKT,),
        in_specs=[
            pl.BlockSpec((D, Q), lambda k: (0, 0)),
            pl.BlockSpec((BK, D), lambda k: (k, 0)),
        ],
        out_specs=pl.BlockSpec((NN, Q), lambda k: (0, 0)),
        out_shape=jax.ShapeDtypeStruct((NN, Q), jnp.int32),
        scratch_shapes=[
            pltpu.VMEM((NN, Q), jnp.float32),
            pltpu.VMEM((NN, Q), jnp.int32),
            pltpu.VMEM((BK, Q), jnp.float32),
        ],
    )(qt, tp_pad)


NC = 2
NS = 16
NW = NC * NS
QPW = Q // NW


def _gather_mean(target_data, idx_flat):
    mesh = plsc.VectorSubcoreMesh(
        core_axis_name="c", subcore_axis_name="s", num_cores=NC,
        num_subcores=NS)

    @functools.partial(
        pl.kernel,
        out_type=jax.ShapeDtypeStruct((Q, D), jnp.float32),
        mesh=mesh,
        scratch_types=[
            pltpu.VMEM((QPW * NN,), jnp.int32),
            pltpu.VMEM((NN, D), jnp.float32),
            pltpu.VMEM((QPW, D), jnp.float32),
            pltpu.SemaphoreType.DMA,
        ],
    )
    def sc_kernel(table_hbm, idx_hbm, out_hbm, idx_v, rows_v, ostage_v, sem):
        wid = lax.axis_index("s") * NC + lax.axis_index("c")
        qbase = wid * QPW
        pltpu.sync_copy(idx_hbm.at[pl.ds(qbase * NN, QPW * NN)], idx_v)

        def per_query(qi, carry):
            off = pl.multiple_of(qi * NN, 8)
            pltpu.async_copy(
                table_hbm.at[idx_v.at[pl.ds(off, NN)]], rows_v, sem).wait()
            for c in range(D // 16):
                def body(r, acc):
                    return acc + rows_v[r, pl.ds(c * 16, 16)]
                acc = lax.fori_loop(0, NN, body, jnp.zeros((16,), jnp.float32))
                ostage_v[qi, pl.ds(c * 16, 16)] = acc * (1.0 / NN)
            return carry

        lax.fori_loop(0, QPW, per_query, 0)
        pltpu.sync_copy(ostage_v, out_hbm.at[pl.ds(qbase, QPW)])

    return sc_kernel(target_data, idx_flat)


def kernel(inputs, target_data):
    tp_pad = jnp.pad(target_data, ((0, K_PAD - K), (0, 0)))
    idx = _topk_indices(inputs.T, tp_pad)
    return _gather_mean(target_data, idx.T.reshape(-1))


# speedup vs baseline: 1.1843x; 1.1309x over previous
import functools

import jax
import jax.numpy as jnp
from jax import lax
from jax.experimental import pallas as pl
from jax.experimental.pallas import tpu as pltpu
from jax.experimental.pallas import tpu_sc as plsc

Q = 1024
D = 128
K = 100000
NN = 32

BK = 2048
CW = 256
NKT = (K + BK - 1) // BK
K_PAD = NKT * BK

_BIG = 3.0e38
_BIGI = 2**31 - 1


def _topk_kernel(qt_ref, tp_ref, oidx_ref, rv_ref, ri_ref, s_ref):
    kt = pl.program_id(0)

    @pl.when(kt == 0)
    def _init():
        rv_ref[...] = jnp.full((NN, Q), _BIG, jnp.float32)
        ri_ref[...] = jnp.full((NN, Q), _BIGI, jnp.int32)

    tpb = tp_ref[...]
    qt = qt_ref[...]
    t2 = jnp.sum(tpb * tpb, axis=1, keepdims=True)
    s = t2 - 2.0 * jnp.dot(tpb, qt, preferred_element_type=jnp.float32)

    row = lax.broadcasted_iota(jnp.int32, (BK, Q), 0)
    s = jnp.where(row + kt * BK >= K, _BIG, s)
    s_ref[...] = s

    slot = lax.broadcasted_iota(jnp.int32, (NN, Q), 0)
    crow = lax.broadcasted_iota(jnp.int32, (CW, Q), 0)

    for c in range(BK // CW):
        base = c * CW + kt * BK
        tau = rv_ref[NN - 1:NN, :]
        sc = s_ref[c * CW:(c + 1) * CW, :]
        cnt = jnp.sum((sc <= tau).astype(jnp.int32), axis=0, keepdims=True)
        need = jnp.max(jnp.minimum(cnt, NN))

        def body(j, carry):
            sv = s_ref[c * CW:(c + 1) * CW, :]
            m = jnp.min(sv, axis=0, keepdims=True)
            sel = jnp.min(jnp.where(sv == m, crow, _BIGI), axis=0,
                          keepdims=True)
            s_ref[c * CW:(c + 1) * CW, :] = jnp.where(crow == sel, _BIG, sv)
            gsel = sel + base
            rv = rv_ref[...]
            ri = ri_ref[...]
            before = (rv < m) | ((rv == m) & (ri < gsel))
            pos = jnp.sum(before.astype(jnp.int32), axis=0, keepdims=True)
            rvs = pltpu.roll(rv, 1, axis=0)
            ris = pltpu.roll(ri, 1, axis=0)
            rv_ref[...] = jnp.where(slot < pos, rv,
                                    jnp.where(slot == pos, m, rvs))
            ri_ref[...] = jnp.where(slot < pos, ri,
                                    jnp.where(slot == pos, gsel, ris))
            return carry

        lax.fori_loop(0, need, body, 0)

    @pl.when(kt == NKT - 1)
    def _out():
        oidx_ref[...] = ri_ref[...]


def _topk_indices(qt, tp_pad):
    return pl.pallas_call(
        _topk_kernel,
        grid=(NKT,),
        in_specs=[
            pl.BlockSpec((D, Q), lambda k: (0, 0)),
            pl.BlockSpec((BK, D), lambda k: (k, 0)),
        ],
        out_specs=pl.BlockSpec((NN, Q), lambda k: (0, 0)),
        out_shape=jax.ShapeDtypeStruct((NN, Q), jnp.int32),
        scratch_shapes=[
            pltpu.VMEM((NN, Q), jnp.float32),
            pltpu.VMEM((NN, Q), jnp.int32),
            pltpu.VMEM((BK, Q), jnp.float32),
        ],
    )(qt, tp_pad)


NC = 2
NS = 16
NW = NC * NS
QPW = Q // NW


def _gather_mean(target_data, idx_flat):
    mesh = plsc.VectorSubcoreMesh(
        core_axis_name="c", subcore_axis_name="s", num_cores=NC,
        num_subcores=NS)

    @functools.partial(
        pl.kernel,
        out_type=jax.ShapeDtypeStruct((Q, D), jnp.float32),
        mesh=mesh,
        scratch_types=[
            pltpu.VMEM((QPW * NN,), jnp.int32),
            pltpu.VMEM((NN, D), jnp.float32),
            pltpu.VMEM((QPW, D), jnp.float32),
            pltpu.SemaphoreType.DMA,
        ],
    )
    def sc_kernel(table_hbm, idx_hbm, out_hbm, idx_v, rows_v, ostage_v, sem):
        wid = lax.axis_index("s") * NC + lax.axis_index("c")
        qbase = wid * QPW
        pltpu.sync_copy(idx_hbm.at[pl.ds(qbase * NN, QPW * NN)], idx_v)

        def per_query(qi, carry):
            off = pl.multiple_of(qi * NN, 8)
            pltpu.async_copy(
                table_hbm.at[idx_v.at[pl.ds(off, NN)]], rows_v, sem).wait()
            for c in range(D // 16):
                def body(r, acc):
                    return acc + rows_v[r, pl.ds(c * 16, 16)]
                acc = lax.fori_loop(0, NN, body, jnp.zeros((16,), jnp.float32))
                ostage_v[qi, pl.ds(c * 16, 16)] = acc * (1.0 / NN)
            return carry

        lax.fori_loop(0, QPW, per_query, 0)
        pltpu.sync_copy(ostage_v, out_hbm.at[pl.ds(qbase, QPW)])

    return sc_kernel(target_data, idx_flat)


def kernel(inputs, target_data):
    tp_pad = jnp.pad(target_data, ((0, K_PAD - K), (0, 0)))
    idx = _topk_indices(inputs.T, tp_pad)
    return _gather_mean(target_data, idx.T.reshape(-1))
